# Initial kernel scaffold; baseline (speedup 1.0000x reference)
import jax
import jax.numpy as jnp
from jax.experimental import pallas as pl


def kernel(hidden_states, orig_weight, orig_bias, new_weight, new_bias):
    raise NotImplementedError("write your pallas kernel here")


# speedup vs baseline: 1.6165x; 1.6165x over previous
import jax
import jax.numpy as jnp
from jax.experimental import pallas as pl

TOP_K = 8
N_EXPERTS = 72
N_PAD = 128
BT = 512


def _router_block(hs_ref, w_ref, b_ref, logits_ref, tw_ref, ti_ref):
    x = hs_ref[...]
    w = w_ref[...]
    logits = jnp.dot(x, w, preferred_element_type=jnp.float32) + b_ref[...]
    logits_ref[...] = logits[:, :N_EXPERTS]

    lanes = jax.lax.broadcasted_iota(jnp.int32, (BT, N_PAD), 1)
    cur = logits
    vals = []
    idxs = []
    for _ in range(TOP_K):
        m = jnp.max(cur, axis=1, keepdims=True)
        amax = jnp.min(jnp.where(cur == m, lanes, N_PAD), axis=1,
                       keepdims=True)
        vals.append(m)
        idxs.append(amax)
        cur = jnp.where(lanes == amax, -jnp.inf, cur)
    v = jnp.concatenate(vals, axis=1)
    i = jnp.concatenate(idxs, axis=1)
    sw = jax.nn.sigmoid(v)
    sw = sw / (jnp.sum(sw, axis=1, keepdims=True) + 1e-8)
    tw_ref[...] = sw
    ti_ref[...] = i


def kernel(hidden_states, orig_weight, orig_bias, new_weight, new_bias):
    b, s, d = hidden_states.shape
    t = b * s
    hs = hidden_states.reshape(t, d)
    all_w = jnp.concatenate([orig_weight, new_weight], axis=0)
    all_b = jnp.concatenate([orig_bias, new_bias], axis=0)
    w_t = jnp.pad(all_w, ((0, N_PAD - N_EXPERTS), (0, 0))).T
    b_pad = jnp.pad(all_b, (0, N_PAD - N_EXPERTS),
                    constant_values=-1e30).reshape(1, N_PAD)

    logits, tw, ti = pl.pallas_call(
        _router_block,
        grid=(t // BT,),
        in_specs=[
            pl.BlockSpec((BT, d), lambda i: (i, 0)),
            pl.BlockSpec((d, N_PAD), lambda i: (0, 0)),
            pl.BlockSpec((1, N_PAD), lambda i: (0, 0)),
        ],
        out_specs=[
            pl.BlockSpec((BT, N_EXPERTS), lambda i: (i, 0)),
            pl.BlockSpec((BT, TOP_K), lambda i: (i, 0)),
            pl.BlockSpec((BT, TOP_K), lambda i: (i, 0)),
        ],
        out_shape=[
            jax.ShapeDtypeStruct((t, N_EXPERTS), jnp.float32),
            jax.ShapeDtypeStruct((t, TOP_K), jnp.float32),
            jax.ShapeDtypeStruct((t, TOP_K), jnp.int32),
        ],
    )(hs, w_t, b_pad)

    return (tw.reshape(b, s, TOP_K),
            ti.reshape(b, s, TOP_K),
            logits.reshape(b, s, N_EXPERTS))


# speedup vs baseline: 1.9223x; 1.1892x over previous
import jax
import jax.numpy as jnp
from jax.experimental import pallas as pl

TOP_K = 8
N_EXPERTS = 72
N_PAD = 128
BT = 512


def _router_block(hs_ref, w_ref, b_ref, logits_ref, tw_ref, ti_ref):
    x = hs_ref[...]
    w = w_ref[...]
    logits = jnp.dot(x, w, preferred_element_type=jnp.float32) + b_ref[...]
    logits_ref[...] = logits[:, :N_EXPERTS]

    lane_desc = (jnp.float32(N_PAD - 1)
                 - jax.lax.broadcasted_iota(jnp.int32, (BT, N_PAD), 1)
                 .astype(jnp.float32))
    neg = jnp.float32(-jnp.inf)
    cur = logits
    vals = []
    encs = []
    for _ in range(TOP_K):
        m = jnp.max(cur, axis=1, keepdims=True)
        enc = jnp.max(jnp.where(cur == m, lane_desc, neg), axis=1,
                      keepdims=True)
        vals.append(m)
        encs.append(enc)
        cur = jnp.where(lane_desc == enc, neg, cur)
    v = jnp.concatenate(vals, axis=1)
    e = jnp.concatenate(encs, axis=1)
    i = (jnp.float32(N_PAD - 1) - e).astype(jnp.int32)
    sw = jax.nn.sigmoid(v)
    sw = sw / (jnp.sum(sw, axis=1, keepdims=True) + 1e-8)
    tw_ref[...] = sw
    ti_ref[...] = i


def kernel(hidden_states, orig_weight, orig_bias, new_weight, new_bias):
    b, s, d = hidden_states.shape
    t = b * s
    hs = hidden_states.reshape(t, d)
    all_w = jnp.concatenate([orig_weight, new_weight], axis=0)
    all_b = jnp.concatenate([orig_bias, new_bias], axis=0)
    w_t = jnp.pad(all_w, ((0, N_PAD - N_EXPERTS), (0, 0))).T
    b_pad = jnp.pad(all_b, (0, N_PAD - N_EXPERTS),
                    constant_values=-1e30).reshape(1, N_PAD)

    logits, tw, ti = pl.pallas_call(
        _router_block,
        grid=(t // BT,),
        in_specs=[
            pl.BlockSpec((BT, d), lambda i: (i, 0)),
            pl.BlockSpec((d, N_PAD), lambda i: (0, 0)),
            pl.BlockSpec((1, N_PAD), lambda i: (0, 0)),
        ],
        out_specs=[
            pl.BlockSpec((BT, N_EXPERTS), lambda i: (i, 0)),
            pl.BlockSpec((BT, TOP_K), lambda i: (i, 0)),
            pl.BlockSpec((BT, TOP_K), lambda i: (i, 0)),
        ],
        out_shape=[
            jax.ShapeDtypeStruct((t, N_EXPERTS), jnp.float32),
            jax.ShapeDtypeStruct((t, TOP_K), jnp.float32),
            jax.ShapeDtypeStruct((t, TOP_K), jnp.int32),
        ],
    )(hs, w_t, b_pad)

    return (tw.reshape(b, s, TOP_K),
            ti.reshape(b, s, TOP_K),
            logits.reshape(b, s, N_EXPERTS))


# speedup vs baseline: 2.4007x; 1.2489x over previous
import jax
import jax.numpy as jnp
from jax.experimental import pallas as pl

TOP_K = 8
N_EXPERTS = 72
BT = 2048


def _router_block(hs_ref, ow_ref, nw_ref, ob_ref, nb_ref,
                  logits_ref, tw_ref, ti_ref):
    x = hs_ref[...]
    w = jnp.concatenate([ow_ref[...], nw_ref[...]], axis=0)
    bias = jnp.concatenate([ob_ref[...], nb_ref[...]], axis=1)
    logits = jax.lax.dot_general(
        x, w, (((1,), (1,)), ((), ())),
        preferred_element_type=jnp.float32) + bias
    logits_ref[...] = logits

    lane_desc = (jnp.float32(127)
                 - jax.lax.broadcasted_iota(jnp.int32, (BT, N_EXPERTS), 1)
                 .astype(jnp.float32))
    neg = jnp.float32(-jnp.inf)
    cur = logits
    vals = []
    encs = []
    for _ in range(TOP_K):
        m = jnp.max(cur, axis=1, keepdims=True)
        enc = jnp.max(jnp.where(cur == m, lane_desc, neg), axis=1,
                      keepdims=True)
        vals.append(m)
        encs.append(enc)
        cur = jnp.where(lane_desc == enc, neg, cur)
    v = jnp.concatenate(vals, axis=1)
    e = jnp.concatenate(encs, axis=1)
    i = (jnp.float32(127) - e).astype(jnp.int32)
    sw = jax.nn.sigmoid(v)
    sw = sw / (jnp.sum(sw, axis=1, keepdims=True) + 1e-8)
    tw_ref[...] = sw
    ti_ref[...] = i


def kernel(hidden_states, orig_weight, orig_bias, new_weight, new_bias):
    b, s, d = hidden_states.shape
    t = b * s
    hs = hidden_states.reshape(t, d)
    ob = orig_bias.reshape(1, -1)
    nb = new_bias.reshape(1, -1)

    logits, tw, ti = pl.pallas_call(
        _router_block,
        grid=(t // BT,),
        in_specs=[
            pl.BlockSpec((BT, d), lambda i: (i, 0)),
            pl.BlockSpec(orig_weight.shape, lambda i: (0, 0)),
            pl.BlockSpec(new_weight.shape, lambda i: (0, 0)),
            pl.BlockSpec(ob.shape, lambda i: (0, 0)),
            pl.BlockSpec(nb.shape, lambda i: (0, 0)),
        ],
        out_specs=[
            pl.BlockSpec((BT, N_EXPERTS), lambda i: (i, 0)),
            pl.BlockSpec((BT, TOP_K), lambda i: (i, 0)),
            pl.BlockSpec((BT, TOP_K), lambda i: (i, 0)),
        ],
        out_shape=[
            jax.ShapeDtypeStruct((t, N_EXPERTS), jnp.float32),
            jax.ShapeDtypeStruct((t, TOP_K), jnp.float32),
            jax.ShapeDtypeStruct((t, TOP_K), jnp.int32),
        ],
    )(hs, orig_weight, new_weight, ob, nb)

    return (tw.reshape(b, s, TOP_K),
            ti.reshape(b, s, TOP_K),
            logits.reshape(b, s, N_EXPERTS))
